# baseline (speedup 1.0000x reference)
import functools

import jax
import jax.numpy as jnp
from jax import lax
from jax.experimental import pallas as pl
from jax.experimental.pallas import tpu as pltpu
from jax.experimental.pallas import tpu_sc as plsc

B = 16384
E = 32
L = 16
W = 128
RPB = W // E

_info = plsc.get_sparse_core_info()
_NC, _NS = _info.num_cores, _info.num_subcores
NW = _NC * _NS
BPW = B // NW
CHUNK = 256
NCHUNK = BPW // CHUNK


def _sc_kernel(user_hbm, item_hbm, ut_hbm, it_hbm, out_hbm,
               uidx_v, iidx_v, ublk_v, iblk_v, urow_v, irow_v, out_v,
               sem_u, sem_i):
    wid = lax.axis_index("s") * _NC + lax.axis_index("c")
    base = wid * BPW
    pltpu.sync_copy(user_hbm.at[pl.ds(base, BPW)], uidx_v)
    pltpu.sync_copy(item_hbm.at[pl.ds(base, BPW)], iidx_v)
    lanes = lax.iota(jnp.int32, L)

    def to_blocks(g, carry):
        ublk_v[pl.ds(g * L, L)] = uidx_v[pl.ds(g * L, L)] >> 2
        iblk_v[pl.ds(g * L, L)] = iidx_v[pl.ds(g * L, L)] >> 2
        return carry

    lax.fori_loop(0, BPW // L, to_blocks, 0)

    for c in range(NCHUNK):
        c0 = c * CHUNK
        cp_u = pltpu.async_copy(
            ut_hbm.at[ublk_v.at[pl.ds(c0, CHUNK)]], urow_v, sem_u)
        cp_i = pltpu.async_copy(
            it_hbm.at[iblk_v.at[pl.ds(c0, CHUNK)]], irow_v, sem_i)
        cp_u.wait()
        cp_i.wait()

        def compute(g, carry):
            rows = g * L + lanes
            ucol = (uidx_v[pl.ds(c0 + g * L, L)] & 3) << 5
            icol = (iidx_v[pl.ds(c0 + g * L, L)] & 3) << 5
            acc = (plsc.load_gather(urow_v, [rows, ucol])
                   * plsc.load_gather(irow_v, [rows, icol]))
            for e in range(1, E):
                acc = acc + (plsc.load_gather(urow_v, [rows, ucol + e])
                             * plsc.load_gather(irow_v, [rows, icol + e]))
            out_v[pl.ds(c0 + g * L, L)] = acc
            return carry

        lax.fori_loop(0, CHUNK // L, compute, 0)

    pltpu.sync_copy(out_v, out_hbm.at[pl.ds(base, BPW)])


@jax.jit
def kernel(user, item, user_table, item_table):
    user = user.astype(jnp.int32)
    item = item.astype(jnp.int32)
    one = jnp.float32(1.0 + 1e-7)
    ut2 = user_table.reshape(250000, W) * one
    it2 = item_table.reshape(250000, W) * one
    mesh = plsc.VectorSubcoreMesh(core_axis_name="c", subcore_axis_name="s")
    f = functools.partial(
        pl.kernel,
        mesh=mesh,
        out_type=jax.ShapeDtypeStruct((B,), jnp.float32),
        compiler_params=pltpu.CompilerParams(needs_layout_passes=False),
        scratch_types=[
            pltpu.VMEM((BPW,), jnp.int32),
            pltpu.VMEM((BPW,), jnp.int32),
            pltpu.VMEM((BPW,), jnp.int32),
            pltpu.VMEM((BPW,), jnp.int32),
            pltpu.VMEM((CHUNK, W), jnp.float32),
            pltpu.VMEM((CHUNK, W), jnp.float32),
            pltpu.VMEM((BPW,), jnp.float32),
            pltpu.SemaphoreType.DMA,
            pltpu.SemaphoreType.DMA,
        ],
    )(_sc_kernel)
    return f(user, item, ut2, it2)


# speedup vs baseline: 1.7484x; 1.7484x over previous
import functools

import jax
import jax.numpy as jnp
from jax import lax
from jax.experimental import pallas as pl
from jax.experimental.pallas import tpu as pltpu
from jax.experimental.pallas import tpu_sc as plsc

B = 16384
E = 32
L = 16
PITCH = 17

_info = plsc.get_sparse_core_info()
_NC, _NS = _info.num_cores, _info.num_subcores
NW = _NC * _NS
BPW = B // NW
CHUNK = 256
NCHUNK = BPW // CHUNK


def _sc_kernel(user_hbm, item_hbm, ut_hbm, it_hbm, out_hbm,
               uidx_v, iidx_v, urow_v, irow_v, out_v, t_v, sem_u, sem_i):
    wid = lax.axis_index("s") * _NC + lax.axis_index("c")
    base = wid * BPW
    pltpu.sync_copy(user_hbm.at[pl.ds(base, BPW)], uidx_v)
    pltpu.sync_copy(item_hbm.at[pl.ds(base, BPW)], iidx_v)
    lanes = lax.iota(jnp.int32, L)

    for c in range(NCHUNK):
        c0 = c * CHUNK

        def fire(g, carry):
            uvec = uidx_v[pl.ds(c0 + g * L, L)]
            ivec = iidx_v[pl.ds(c0 + g * L, L)]
            for j in range(L):
                i = g * L + j
                pltpu.make_async_copy(
                    ut_hbm.at[pl.ds(uvec[j], 1), :],
                    urow_v.at[pl.ds(i, 1), :], sem_u).start()
                pltpu.make_async_copy(
                    it_hbm.at[pl.ds(ivec[j], 1), :],
                    irow_v.at[pl.ds(i, 1), :], sem_i).start()
            return carry

        lax.fori_loop(0, CHUNK // L, fire, 0)
        pltpu.make_async_copy(ut_hbm.at[pl.ds(0, CHUNK), :], urow_v,
                              sem_u).wait()
        pltpu.make_async_copy(it_hbm.at[pl.ds(0, CHUNK), :], irow_v,
                              sem_i).wait()

        def compute(g, carry):
            for j in range(L):
                i = g * L + j
                u1 = urow_v[i, pl.ds(0, L)]
                u2 = urow_v[i, pl.ds(L, L)]
                v1 = irow_v[i, pl.ds(0, L)]
                v2 = irow_v[i, pl.ds(L, L)]
                s = u1 * v1 + u2 * v2
                plsc.store_scatter(t_v, [lanes * PITCH + j], s)
            acc = t_v[pl.ds(0, L)]
            for l in range(1, L):
                acc = acc + t_v[pl.ds(l * PITCH, L)]
            out_v[pl.ds(c0 + g * L, L)] = acc
            return carry

        lax.fori_loop(0, CHUNK // L, compute, 0)

    pltpu.sync_copy(out_v, out_hbm.at[pl.ds(base, BPW)])


@jax.jit
def kernel(user, item, user_table, item_table):
    user = user.astype(jnp.int32)
    item = item.astype(jnp.int32)
    mesh = plsc.VectorSubcoreMesh(core_axis_name="c", subcore_axis_name="s")
    f = functools.partial(
        pl.kernel,
        mesh=mesh,
        out_type=jax.ShapeDtypeStruct((B,), jnp.float32),
        compiler_params=pltpu.CompilerParams(needs_layout_passes=False),
        scratch_types=[
            pltpu.VMEM((BPW,), jnp.int32),
            pltpu.VMEM((BPW,), jnp.int32),
            pltpu.VMEM((CHUNK, E), jnp.float32),
            pltpu.VMEM((CHUNK, E), jnp.float32),
            pltpu.VMEM((BPW,), jnp.float32),
            pltpu.VMEM((L * PITCH,), jnp.float32),
            pltpu.SemaphoreType.DMA,
            pltpu.SemaphoreType.DMA,
        ],
    )(_sc_kernel)
    return f(user, item, user_table, item_table)


# speedup vs baseline: 3.8907x; 2.2253x over previous
import functools

import jax
import jax.numpy as jnp
from jax import lax
from jax.experimental import pallas as pl
from jax.experimental.pallas import tpu as pltpu
from jax.experimental.pallas import tpu_sc as plsc

B = 16384
E = 32
V = 1000000
L = 16
W = 128
PITCH = 17
CH = 8

_info = plsc.get_sparse_core_info()
_NC, _NS = _info.num_cores, _info.num_subcores
NW = _NC * _NS
BPW = B // NW


def _sc_kernel(user_hbm, item_hbm, ut_hbm, it_hbm, out_hbm,
               uidx_v, iidx_v, ubuf_v, ibuf_v, out_v, t_v, sem_u, sem_i):
    wid = lax.axis_index("s") * _NC + lax.axis_index("c")
    base = wid * BPW
    pltpu.sync_copy(user_hbm.at[pl.ds(base, BPW)], uidx_v)
    pltpu.sync_copy(item_hbm.at[pl.ds(base, BPW)], iidx_v)
    lanes = lax.iota(jnp.int32, L)

    def group(g, carry):
        uvec = uidx_v[pl.ds(g * L, L)]
        ivec = iidx_v[pl.ds(g * L, L)]
        for sub in range(2):
            for jj in range(CH):
                j = sub * CH + jj
                su = pl.multiple_of((uvec[j] >> 7) * W, W)
                si = pl.multiple_of((ivec[j] >> 7) * W, W)
                pltpu.make_async_copy(
                    ut_hbm.at[:, pl.ds(su, W)], ubuf_v.at[jj], sem_u).start()
                pltpu.make_async_copy(
                    it_hbm.at[:, pl.ds(si, W)], ibuf_v.at[jj], sem_i).start()
            for jj in range(CH):
                pltpu.make_async_copy(
                    ut_hbm.at[:, pl.ds(0, W)], ubuf_v.at[jj], sem_u).wait()
                pltpu.make_async_copy(
                    it_hbm.at[:, pl.ds(0, W)], ibuf_v.at[jj], sem_i).wait()
            for jj in range(CH):
                j = sub * CH + jj
                ucol = jnp.full((L,), 0, jnp.int32) + (uvec[j] & 127)
                icol = jnp.full((L,), 0, jnp.int32) + (ivec[j] & 127)
                kk = jnp.full((L,), jj, jnp.int32)
                u_lo = plsc.load_gather(ubuf_v, [kk, lanes, ucol])
                u_hi = plsc.load_gather(ubuf_v, [kk, lanes + L, ucol])
                v_lo = plsc.load_gather(ibuf_v, [kk, lanes, icol])
                v_hi = plsc.load_gather(ibuf_v, [kk, lanes + L, icol])
                s = u_lo * v_lo + u_hi * v_hi
                plsc.store_scatter(t_v, [lanes * PITCH + j], s)
        acc = t_v[pl.ds(0, L)]
        for l in range(1, L):
            acc = acc + t_v[pl.ds(l * PITCH, L)]
        out_v[pl.ds(g * L, L)] = acc
        return carry

    lax.fori_loop(0, BPW // L, group, 0)
    pltpu.sync_copy(out_v, out_hbm.at[pl.ds(base, BPW)])


@jax.jit
def kernel(user, item, user_table, item_table):
    user = user.astype(jnp.int32)
    item = item.astype(jnp.int32)
    utT = user_table.T
    itT = item_table.T
    mesh = plsc.VectorSubcoreMesh(core_axis_name="c", subcore_axis_name="s")
    f = functools.partial(
        pl.kernel,
        mesh=mesh,
        out_type=jax.ShapeDtypeStruct((B,), jnp.float32),
        compiler_params=pltpu.CompilerParams(needs_layout_passes=False),
        scratch_types=[
            pltpu.VMEM((BPW,), jnp.int32),
            pltpu.VMEM((BPW,), jnp.int32),
            pltpu.VMEM((CH, E, W), jnp.float32),
            pltpu.VMEM((CH, E, W), jnp.float32),
            pltpu.VMEM((BPW,), jnp.float32),
            pltpu.VMEM((L * PITCH,), jnp.float32),
            pltpu.SemaphoreType.DMA,
            pltpu.SemaphoreType.DMA,
        ],
    )(_sc_kernel)
    return f(user, item, utT, itT)
